# Initial kernel scaffold; baseline (speedup 1.0000x reference)
import jax
import jax.numpy as jnp
from jax.experimental import pallas as pl


def kernel(input_ids, seg_ids, tok_table, seg_table, pos_table):
    raise NotImplementedError("write your pallas kernel here")


# speedup vs baseline: 1.0938x; 1.0938x over previous
import functools

import jax
import jax.numpy as jnp
from jax import lax
from jax.experimental import pallas as pl
from jax.experimental.pallas import tpu as pltpu
from jax.experimental.pallas import tpu_sc as plsc

NC = 2
NS = 16
NW = NC * NS
L = 16


def _emb_body(n_batch, seq_len, d_model,
              ids_hbm, seg_hbm, tok_hbm, segtab_hbm, pos_hbm, out_hbm,
              pos_buf, tok_buf, seg_rows, diff_v, idx_v, segi_v, segf_v, sem):
    p_per_w = seq_len // NW
    nj = d_model // L
    wid = lax.axis_index("s") * NC + lax.axis_index("c")
    p0 = wid * p_per_w

    pltpu.sync_copy(pos_hbm.at[pl.ds(p0, p_per_w)], pos_buf)
    pltpu.sync_copy(segtab_hbm, seg_rows)

    def pre_j(j, carry):
        sl = pl.ds(j * L, L)
        s0 = seg_rows[0, sl]
        diff_v[sl] = seg_rows[1, sl] - s0

        def pre_i(i, c):
            plsc.addupdate(pos_buf.at[i, sl], s0)
            return c

        return lax.fori_loop(0, p_per_w, pre_i, carry, unroll=4)

    lax.fori_loop(0, nj, pre_j, 0)

    def batch_body(b, carry):
        base = b * seq_len + p0
        pltpu.sync_copy(ids_hbm.at[pl.ds(base, p_per_w)], idx_v)
        pltpu.sync_copy(seg_hbm.at[pl.ds(base, p_per_w)], segi_v)

        def conv(jj, c):
            sl = pl.ds(jj * L, L)
            segf_v[sl] = segi_v[sl].astype(jnp.float32)
            return c

        lax.fori_loop(0, p_per_w // L, conv, 0)

        pltpu.async_copy(tok_hbm.at[idx_v], tok_buf, sem).wait()

        def row_i(i, c):
            s = plsc.load_gather(segf_v, [jnp.broadcast_to(i, (L,))])

            def col_j(j, cc):
                sl = pl.ds(j * L, L)
                plsc.addupdate(tok_buf.at[i, sl],
                               pos_buf[i, sl] + s * diff_v[sl])
                return cc

            return lax.fori_loop(0, nj, col_j, c, unroll=4)

        lax.fori_loop(0, p_per_w, row_i, 0)
        pltpu.sync_copy(tok_buf, out_hbm.at[pl.ds(base, p_per_w)])
        return carry

    lax.fori_loop(0, n_batch, batch_body, 0)


@functools.cache
def _build(n_batch, seq_len, d_model, vocab, n_seg, maxlen):
    assert seq_len % NW == 0 and d_model % L == 0
    p_per_w = seq_len // NW
    assert p_per_w % L == 0
    mesh = plsc.VectorSubcoreMesh(core_axis_name="c", subcore_axis_name="s",
                                  num_cores=NC, num_subcores=NS)
    body = functools.partial(_emb_body, n_batch, seq_len, d_model)
    return pl.kernel(
        body,
        out_type=jax.ShapeDtypeStruct((n_batch * seq_len, d_model),
                                      jnp.float32),
        mesh=mesh,
        scratch_types=[
            pltpu.VMEM((p_per_w, d_model), jnp.float32),
            pltpu.VMEM((p_per_w, d_model), jnp.float32),
            pltpu.VMEM((2, d_model), jnp.float32),
            pltpu.VMEM((d_model,), jnp.float32),
            pltpu.VMEM((p_per_w,), jnp.int32),
            pltpu.VMEM((p_per_w,), jnp.int32),
            pltpu.VMEM((p_per_w,), jnp.float32),
            pltpu.SemaphoreType.DMA,
        ],
        compiler_params=pltpu.CompilerParams(needs_layout_passes=False),
        name="sc_embedding_lookup",
    )


def kernel(input_ids, seg_ids, tok_table, seg_table, pos_table):
    n_batch, seq_len = input_ids.shape
    vocab, d_model = tok_table.shape
    fn = _build(n_batch, seq_len, d_model, vocab,
                seg_table.shape[0], pos_table.shape[0])
    out = fn(input_ids.reshape(-1), seg_ids.reshape(-1),
             tok_table, seg_table, pos_table)
    return out.reshape(n_batch, seq_len, d_model)
